# baseline (speedup 1.0000x reference)
import functools

import jax
import jax.numpy as jnp
from jax import lax
from jax.experimental import pallas as pl
from jax.experimental.pallas import tpu as pltpu
from jax.experimental.pallas import tpu_sc as plsc

N = 10000
E = 320000
D = 128
NC = 2
NS = 16
NW = NC * NS
EPW = E // NW
K = 80
NCHUNK = EPW // K
NB = 4
RPT = 624
REM = N - RPT * NS


def _sc_partial(data, se, de, zeros):
    mesh = plsc.VectorSubcoreMesh(
        core_axis_name="c", subcore_axis_name="s", num_cores=NC
    )

    @functools.partial(
        pl.kernel,
        out_type=jax.ShapeDtypeStruct((NC, N, D), jnp.float32),
        mesh=mesh,
        scratch_types=[pltpu.VMEM_SHARED((N, D), jnp.float32)]
        + [pltpu.VMEM((K,), jnp.int32) for _ in range(2 * NB)]
        + [pltpu.VMEM((K, D), jnp.float32) for _ in range(NB)]
        + [pltpu.SemaphoreType.DMA for _ in range(2 * NB)],
    )
    def k(data_hbm, se_hbm, de_hbm, zero_hbm, out_hbm, acc, *scr):
        srcb = scr[0:NB]
        dstb = scr[NB:2 * NB]
        rows = scr[2 * NB:3 * NB]
        isem = scr[3 * NB:4 * NB]
        gsem = scr[4 * NB:5 * NB]
        c = lax.axis_index("c")
        s = lax.axis_index("s")
        wid = s * NC + c

        pltpu.sync_copy(
            zero_hbm.at[pl.ds(s * RPT, RPT)], acc.at[pl.ds(s * RPT, RPT)]
        )

        @pl.when(s == 0)
        def _zero_rem():
            pltpu.sync_copy(
                zero_hbm.at[pl.ds(RPT * NS, REM)], acc.at[pl.ds(RPT * NS, REM)]
            )

        plsc.subcore_barrier()

        base0 = wid * EPW

        def start_idx(g, b):
            pltpu.async_copy(se_hbm.at[pl.ds(base0 + g * K, K)], srcb[b], isem[b])
            pltpu.async_copy(de_hbm.at[pl.ds(base0 + g * K, K)], dstb[b], isem[b])

        def wait_idx(g, b):
            pltpu.make_async_copy(
                se_hbm.at[pl.ds(base0 + g * K, K)], srcb[b], isem[b]
            ).wait()
            pltpu.make_async_copy(
                de_hbm.at[pl.ds(base0 + g * K, K)], dstb[b], isem[b]
            ).wait()

        def start_gather(b):
            pltpu.async_copy(data_hbm.at[srcb[b]], rows[b], gsem[b])

        def wait_gather(b):
            pltpu.make_async_copy(data_hbm.at[srcb[b]], rows[b], gsem[b]).wait()

        def scatter(b):
            pltpu.sync_copy(rows[b], acc.at[dstb[b]], add=True)

        for b in range(NB):
            start_idx(b, b)
        for b in range(NB - 1):
            wait_idx(b, b)
            start_gather(b)

        MAIN = 120

        @pl.loop(0, MAIN, step=NB)
        def _grp(g0):
            for b in range(NB):
                g = g0 + b
                b3 = (b + 3) % NB
                wait_gather(b)
                wait_idx(g + 3, b3)
                start_gather(b3)
                scatter(b)
                start_idx(g + NB, b)

        for g in range(MAIN, NCHUNK):
            b = g % NB
            wait_gather(b)
            if g + 3 < NCHUNK:
                b3 = (g + 3) % NB
                wait_idx(g + 3, b3)
                start_gather(b3)
            scatter(b)
            if g + NB < NCHUNK:
                start_idx(g + NB, b)

        plsc.subcore_barrier()
        pltpu.sync_copy(
            acc.at[pl.ds(s * RPT, RPT)], out_hbm.at[c, pl.ds(s * RPT, RPT)]
        )

        @pl.when(s == 0)
        def _flush_rem():
            pltpu.sync_copy(
                acc.at[pl.ds(RPT * NS, REM)], out_hbm.at[c, pl.ds(RPT * NS, REM)]
            )

    return k(data, se, de, zeros)


def _combine(partial):
    def body(p_ref, o_ref):
        o_ref[...] = p_ref[0] + p_ref[1]

    return pl.pallas_call(
        body,
        out_shape=jax.ShapeDtypeStruct((N, D), jnp.float32),
        grid=(10,),
        in_specs=[pl.BlockSpec((2, 1000, D), lambda i: (0, i, 0))],
        out_specs=pl.BlockSpec((1000, D), lambda i: (i, 0)),
    )(partial)


@jax.jit
def kernel(data, edge_index):
    se = edge_index[0]
    de = edge_index[1]
    zeros = jnp.zeros((N, D), jnp.float32)
    partial = _sc_partial(data, se, de, zeros)
    return _combine(partial)


# speedup vs baseline: 1.2883x; 1.2883x over previous
import functools

import jax
import jax.numpy as jnp
from jax import lax
from jax.experimental import pallas as pl
from jax.experimental.pallas import tpu as pltpu
from jax.experimental.pallas import tpu_sc as plsc

N = 10000
E = 320000
D = 128
NC = 2
NS = 16
NW = NC * NS
EPW = E // NW
K = 80
NCHUNK = EPW // K
NB = 4
NI = 8
MAIN = 120
RPT = 624
REM = N - RPT * NS


def _sc_partial(data, se, de, zeros):
    mesh = plsc.VectorSubcoreMesh(
        core_axis_name="c", subcore_axis_name="s", num_cores=NC
    )

    @functools.partial(
        pl.kernel,
        out_type=jax.ShapeDtypeStruct((NC, N, D), jnp.float32),
        mesh=mesh,
        scratch_types=[pltpu.VMEM_SHARED((N, D), jnp.float32)]
        + [pltpu.VMEM((K,), jnp.int32) for _ in range(2 * NI)]
        + [pltpu.VMEM((K, D), jnp.float32) for _ in range(NB)]
        + [pltpu.SemaphoreType.DMA for _ in range(NI + 2 * NB)],
    )
    def k(data_hbm, se_hbm, de_hbm, zero_hbm, out_hbm, acc, *scr):
        srcb = scr[0:NI]
        dstb = scr[NI:2 * NI]
        rows = scr[2 * NI:2 * NI + NB]
        isem = scr[2 * NI + NB:2 * NI + NB + NI]
        gsem = scr[2 * NI + NB + NI:2 * NI + NB + NI + NB]
        ssem = scr[2 * NI + NB + NI + NB:]
        c = lax.axis_index("c")
        s = lax.axis_index("s")
        wid = s * NC + c

        pltpu.sync_copy(
            zero_hbm.at[pl.ds(s * RPT, RPT)], acc.at[pl.ds(s * RPT, RPT)]
        )

        @pl.when(s == 0)
        def _zero_rem():
            pltpu.sync_copy(
                zero_hbm.at[pl.ds(RPT * NS, REM)], acc.at[pl.ds(RPT * NS, REM)]
            )

        plsc.subcore_barrier()

        base0 = wid * EPW

        def start_idx(g, bi):
            pltpu.async_copy(se_hbm.at[pl.ds(base0 + g * K, K)], srcb[bi], isem[bi])
            pltpu.async_copy(de_hbm.at[pl.ds(base0 + g * K, K)], dstb[bi], isem[bi])

        def wait_idx(g, bi):
            pltpu.make_async_copy(
                se_hbm.at[pl.ds(base0 + g * K, K)], srcb[bi], isem[bi]
            ).wait()
            pltpu.make_async_copy(
                de_hbm.at[pl.ds(base0 + g * K, K)], dstb[bi], isem[bi]
            ).wait()

        def start_gather(b, bi):
            pltpu.async_copy(data_hbm.at[srcb[bi]], rows[b], gsem[b])

        def wait_gather(b, bi):
            pltpu.make_async_copy(data_hbm.at[srcb[bi]], rows[b], gsem[b]).wait()

        def start_scatter(b, bi):
            pltpu.async_copy(rows[b], acc.at[dstb[bi]], ssem[b], add=True)

        def wait_scatter(b, bi):
            pltpu.make_async_copy(rows[b], acc.at[dstb[bi]], ssem[b]).wait()

        for g in range(NB):
            start_idx(g, g % NI)
        for g in range(NB - 1):
            wait_idx(g, g % NI)
            start_gather(g % NB, g % NI)

        for g in range(NI):
            wait_gather(g % NB, g % NI)
            if g > 0:
                wait_scatter((g - 1) % NB, (g - 1) % NI)
            wait_idx(g + 3, (g + 3) % NI)
            start_gather((g + 3) % NB, (g + 3) % NI)
            start_scatter(g % NB, g % NI)
            start_idx(g + NB, (g + NB) % NI)

        @pl.loop(NI, MAIN, step=NI)
        def _grp(g0):
            for j in range(NI):
                g = g0 + j
                wait_gather(j % NB, j)
                wait_scatter((j - 1) % NB, (j - 1) % NI)
                wait_idx(g + 3, (j + 3) % NI)
                start_gather((j + 3) % NB, (j + 3) % NI)
                start_scatter(j % NB, j)
                start_idx(g + NB, (j + NB) % NI)

        for g in range(MAIN, NCHUNK):
            b = g % NB
            bi = g % NI
            wait_gather(b, bi)
            wait_scatter((g - 1) % NB, (g - 1) % NI)
            if g + 3 < NCHUNK:
                wait_idx(g + 3, (g + 3) % NI)
                start_gather((g + 3) % NB, (g + 3) % NI)
            start_scatter(b, bi)
            if g + NB < NCHUNK:
                start_idx(g + NB, (g + NB) % NI)
        wait_scatter((NCHUNK - 1) % NB, (NCHUNK - 1) % NI)

        plsc.subcore_barrier()
        pltpu.sync_copy(
            acc.at[pl.ds(s * RPT, RPT)], out_hbm.at[c, pl.ds(s * RPT, RPT)]
        )

        @pl.when(s == 0)
        def _flush_rem():
            pltpu.sync_copy(
                acc.at[pl.ds(RPT * NS, REM)], out_hbm.at[c, pl.ds(RPT * NS, REM)]
            )

    return k(data, se, de, zeros)


def _combine(partial):
    def body(p_ref, o_ref):
        o_ref[...] = p_ref[0] + p_ref[1]

    return pl.pallas_call(
        body,
        out_shape=jax.ShapeDtypeStruct((N, D), jnp.float32),
        grid=(10,),
        in_specs=[pl.BlockSpec((2, 1000, D), lambda i: (0, i, 0))],
        out_specs=pl.BlockSpec((1000, D), lambda i: (i, 0)),
    )(partial)


@jax.jit
def kernel(data, edge_index):
    se = edge_index[0]
    de = edge_index[1]
    zeros = jnp.zeros((N, D), jnp.float32)
    partial = _sc_partial(data, se, de, zeros)
    return _combine(partial)
